# baseline (speedup 1.0000x reference)
import functools

import jax
import jax.numpy as jnp
import numpy as np
from jax import lax
from jax.experimental import pallas as pl
from jax.experimental.pallas import tpu as pltpu
from jax.experimental.pallas import tpu_sc as plsc

N_NODES = 10000
D_IN = 128
N_EDGES = 320000
NUM_CORES = 2
NUM_TILES = 16
CHUNK = 64
K_CHUNKS = 320
SB = 8
NB = K_CHUNKS // SB
GR = 4
PT = K_CHUNKS * CHUNK
E_PAD = NUM_TILES * PT
ROWS_PER_TILE = N_NODES // NUM_TILES

UNPACK_PERM = np.empty((D_IN,), np.int64)
for _g in range(D_IN // 32):
  for _k in range(16):
    UNPACK_PERM[32 * _g + _k] = 32 * _g + 2 * _k
    UNPACK_PERM[32 * _g + 16 + _k] = 32 * _g + 2 * _k + 1


def _sc_aggregate(tbl, eidx, wts):
  mesh = plsc.VectorSubcoreMesh(core_axis_name="c", subcore_axis_name="s")

  @functools.partial(
      pl.kernel,
      mesh=mesh,
      compiler_params=pltpu.CompilerParams(use_tc_tiling_on_sc=False,
                                           needs_layout_passes=False),
      out_type=(
          jax.ShapeDtypeStruct((NUM_CORES, N_NODES, D_IN), jnp.float32),
          jax.ShapeDtypeStruct((NUM_CORES, N_NODES, 16), jnp.float32),
      ),
      scratch_types=[
          pltpu.VMEM((2, SB, 2, CHUNK), jnp.int32),
          pltpu.VMEM((2, SB, CHUNK), jnp.float32),
          pltpu.VMEM((GR, CHUNK, D_IN), jnp.bfloat16),
          pltpu.VMEM((2, CHUNK, D_IN), jnp.float32),
          pltpu.VMEM((2, CHUNK, 16), jnp.float32),
          pltpu.VMEM_SHARED((N_NODES, D_IN), jnp.float32),
          pltpu.VMEM_SHARED((N_NODES, 16), jnp.float32),
          pltpu.SemaphoreType.DMA((GR,)),
          pltpu.SemaphoreType.DMA((2,)),
          pltpu.SemaphoreType.DMA((2,)),
          pltpu.SemaphoreType.DMA((2,)),
          pltpu.SemaphoreType.DMA((2,)),
      ],
  )
  def k(tbl_h, eidx_h, wts_h, out_h, wout_h, idx_v, w_v, gbuf, sbuf, wrow,
        acc, wacc, gsem, ssem, wssem, stsem, wstsem):
    c = lax.axis_index("c")
    s = lax.axis_index("s")
    base = s * ROWS_PER_TILE

    def zrow(i, carry):
      for d in range(D_IN // 16):
        sbuf[0, i, pl.ds(d * 16, 16)] = jnp.zeros((16,), jnp.float32)
      wrow[0, i, pl.ds(0, 16)] = jnp.zeros((16,), jnp.float32)
      return carry
    lax.fori_loop(0, CHUNK, zrow, 0)
    for t in range(ROWS_PER_TILE // CHUNK):
      pltpu.sync_copy(sbuf.at[0], acc.at[pl.ds(base + t * CHUNK, CHUNK)])
      pltpu.sync_copy(wrow.at[0], wacc.at[pl.ds(base + t * CHUNK, CHUNK)])
    rem = ROWS_PER_TILE % CHUNK
    if rem:
      off = base + ROWS_PER_TILE - rem
      pltpu.sync_copy(sbuf.at[0, pl.ds(0, rem)], acc.at[pl.ds(off, rem)])
      pltpu.sync_copy(wrow.at[0, pl.ds(0, rem)], wacc.at[pl.ds(off, rem)])
    plsc.subcore_barrier()

    pltpu.sync_copy(eidx_h.at[c, s, pl.ds(0, SB)], idx_v.at[0])
    pltpu.sync_copy(wts_h.at[s, pl.ds(0, SB)], w_v.at[0])
    pltpu.async_copy(eidx_h.at[c, s, pl.ds(SB, SB)], idx_v.at[1],
                     stsem.at[1])
    pltpu.async_copy(wts_h.at[s, pl.ds(SB, SB)], w_v.at[1], wstsem.at[1])
    for q in range(GR):
      pltpu.async_copy(tbl_h.at[idx_v.at[0, q, 0]], gbuf.at[q], gsem.at[q])

    def body(j, carry):
      p4 = lax.rem(j, GR)
      sp = lax.rem(j, 2)
      jj = lax.rem(j, SB)
      bb = lax.rem(j // SB, 2)

      pltpu.make_async_copy(tbl_h.at[idx_v.at[bb, jj, 0]], gbuf.at[p4],
                            gsem.at[p4]).wait()

      @pl.when(j >= 2)
      def _():
        jm = j - 2
        bm = lax.rem(jm // SB, 2)
        jjm = lax.rem(jm, SB)
        pltpu.make_async_copy(sbuf.at[sp], acc.at[idx_v.at[bm, jjm, 1]],
                              ssem.at[sp]).wait()
        pltpu.make_async_copy(wrow.at[sp], wacc.at[idx_v.at[bm, jjm, 1]],
                              wssem.at[sp]).wait()

      def mul(g, carry2):
        wv = w_v[bb, jj, pl.ds(g * 16, 16)]
        for e16 in range(16):
          wsc = wv[e16]
          row = g * 16 + e16
          wsplat = jnp.broadcast_to(wsc, (16,))
          wb32 = plsc.pack(wsplat, wsplat, format=plsc.PackFormat.INTERLEAVED)
          for h in range(D_IN // 32):
            v32 = gbuf[p4, row, pl.ds(h * 32, 32)] * wb32
            va, vb = plsc.unpack(v32, format=plsc.PackFormat.INTERLEAVED)
            sbuf[sp, row, pl.ds(h * 32, 16)] = va
            sbuf[sp, row, pl.ds(h * 32 + 16, 16)] = vb
          wrow[sp, row, pl.ds(0, 16)] = wsplat
        return carry2
      lax.fori_loop(0, CHUNK // 16, mul, 0)

      pltpu.async_copy(sbuf.at[sp], acc.at[idx_v.at[bb, jj, 1]],
                       ssem.at[sp], add=True)
      pltpu.async_copy(wrow.at[sp], wacc.at[idx_v.at[bb, jj, 1]],
                       wssem.at[sp], add=True)

      @pl.when((jj == 1) & (j > SB) & (j // SB + 1 < NB))
      def _():
        nblk = j // SB + 1
        tb = lax.rem(nblk, 2)
        pltpu.async_copy(eidx_h.at[c, s, pl.ds(nblk * SB, SB)],
                         idx_v.at[tb], stsem.at[tb])
        pltpu.async_copy(wts_h.at[s, pl.ds(nblk * SB, SB)],
                         w_v.at[tb], wstsem.at[tb])

      @pl.when(j + GR < K_CHUNKS)
      def _():
        jn = j + GR
        nb = lax.rem(jn // SB, 2)

        @pl.when(lax.rem(jn, SB) == 0)
        def _():
          pltpu.make_async_copy(eidx_h.at[c, s, pl.ds(jn, SB)],
                                idx_v.at[nb], stsem.at[nb]).wait()
          pltpu.make_async_copy(wts_h.at[s, pl.ds(jn, SB)],
                                w_v.at[nb], wstsem.at[nb]).wait()

        pltpu.async_copy(tbl_h.at[idx_v.at[nb, lax.rem(jn, SB), 0]],
                         gbuf.at[p4], gsem.at[p4])
      return carry
    lax.fori_loop(0, K_CHUNKS, body, 0)

    for jl in (K_CHUNKS - 2, K_CHUNKS - 1):
      sp = jl % 2
      bl = (jl // SB) % 2
      jjl = jl % SB
      pltpu.make_async_copy(sbuf.at[sp], acc.at[idx_v.at[bl, jjl, 1]],
                            ssem.at[sp]).wait()
      pltpu.make_async_copy(wrow.at[sp], wacc.at[idx_v.at[bl, jjl, 1]],
                            wssem.at[sp]).wait()
    plsc.subcore_barrier()
    pltpu.sync_copy(acc.at[pl.ds(base, ROWS_PER_TILE)],
                    out_h.at[c, pl.ds(base, ROWS_PER_TILE)])
    pltpu.sync_copy(wacc.at[pl.ds(base, ROWS_PER_TILE)],
                    wout_h.at[c, pl.ds(base, ROWS_PER_TILE)])

  return k(tbl, eidx, wts)


def _tc_dense(agg, wagg, feat, w1t, w2t, a1, a2, whht, bih, bhh):
  blk = 1000
  grid = (N_NODES // blk,)

  def body(agg_ref, wagg_ref, feat_ref, w1_ref, w2_ref, a1_ref, a2_ref,
           whh_ref, bih_ref, bhh_ref, out_ref):
    m1 = agg_ref[0]
    m2 = agg_ref[1]
    ws1 = wagg_ref[0][:, :1]
    ws2 = wagg_ref[1][:, :1]
    neigh1 = jnp.where(ws1 > 0, m1 / jnp.where(ws1 > 0, ws1, 1.0), 0.0)
    neigh2 = jnp.where(ws2 > 0, m2 / jnp.where(ws2 > 0, ws2, 1.0), 0.0)
    dot = functools.partial(jnp.dot, precision=lax.Precision.HIGHEST,
                            preferred_element_type=jnp.float32)
    n1 = dot(neigh1, w1_ref[...])
    n2 = dot(neigh2, w2_ref[...])
    gi = dot(n1, a1_ref[...]) + dot(n2, a2_ref[...]) + bih_ref[...]
    ft = feat_ref[...]
    gh = dot(ft, whh_ref[...]) + bhh_ref[...]
    r = jax.nn.sigmoid(gi[:, :D_IN] + gh[:, :D_IN])
    z = jax.nn.sigmoid(gi[:, D_IN:2 * D_IN] + gh[:, D_IN:2 * D_IN])
    n = jnp.tanh(gi[:, 2 * D_IN:] + r * gh[:, 2 * D_IN:])
    out_ref[...] = (1.0 - z) * n + z * ft

  return pl.pallas_call(
      body,
      grid=grid,
      in_specs=[
          pl.BlockSpec((NUM_CORES, blk, D_IN), lambda i: (0, i, 0)),
          pl.BlockSpec((NUM_CORES, blk, 16), lambda i: (0, i, 0)),
          pl.BlockSpec((blk, D_IN), lambda i: (i, 0)),
          pl.BlockSpec((D_IN, D_IN), lambda i: (0, 0)),
          pl.BlockSpec((D_IN, D_IN), lambda i: (0, 0)),
          pl.BlockSpec((D_IN, 3 * D_IN), lambda i: (0, 0)),
          pl.BlockSpec((D_IN, 3 * D_IN), lambda i: (0, 0)),
          pl.BlockSpec((D_IN, 3 * D_IN), lambda i: (0, 0)),
          pl.BlockSpec((1, 3 * D_IN), lambda i: (0, 0)),
          pl.BlockSpec((1, 3 * D_IN), lambda i: (0, 0)),
      ],
      out_specs=pl.BlockSpec((blk, D_IN), lambda i: (i, 0)),
      out_shape=jax.ShapeDtypeStruct((N_NODES, D_IN), jnp.float32),
  )(agg, wagg, feat, w1t, w2t, a1, a2, whht, bih, bhh)


@jax.jit
def kernel(feat, edge_index, edge_weight, W1, W2, W_ih, W_hh, b_ih, b_hh):
  pad = E_PAD - N_EDGES
  src = jnp.concatenate([edge_index[0], jnp.zeros((pad,), jnp.int32)])
  dst = jnp.concatenate([edge_index[1], jnp.zeros((pad,), jnp.int32)])
  w = jnp.concatenate([edge_weight, jnp.zeros((pad,), jnp.float32)])
  src_r = src.reshape(NUM_TILES, K_CHUNKS, CHUNK)
  dst_r = dst.reshape(NUM_TILES, K_CHUNKS, CHUNK)
  wts = w.reshape(NUM_TILES, K_CHUNKS, CHUNK)
  eidx = jnp.stack([jnp.stack([src_r, dst_r], axis=2),
                    jnp.stack([dst_r, src_r], axis=2)])
  tbl = feat.astype(jnp.bfloat16)

  tbl, eidx, wts = lax.optimization_barrier((tbl, eidx, wts))
  agg, wagg = _sc_aggregate(tbl, eidx, wts)

  w1t = W1.T[UNPACK_PERM]
  w2t = W2.T[UNPACK_PERM]
  wiht = W_ih.T
  a1 = wiht[:D_IN]
  a2 = wiht[D_IN:]
  whht = W_hh.T
  bih = b_ih.reshape(1, 3 * D_IN)
  bhh = b_hh.reshape(1, 3 * D_IN)
  return _tc_dense(agg, wagg, feat, w1t, w2t, a1, a2, whht, bih, bhh)


# speedup vs baseline: 1.0888x; 1.0888x over previous
import functools

import jax
import jax.numpy as jnp
import numpy as np
from jax import lax
from jax.experimental import pallas as pl
from jax.experimental.pallas import tpu as pltpu
from jax.experimental.pallas import tpu_sc as plsc

N_NODES = 10000
D_IN = 128
D_ACC = 144
N_EDGES = 320000
NUM_CORES = 2
NUM_TILES = 16
CHUNK = 64
BLK = 4
K_CHUNKS = 320
NITER = K_CHUNKS // BLK
PT = K_CHUNKS * CHUNK
E_PAD = NUM_TILES * PT
ROWS_PER_TILE = N_NODES // NUM_TILES

UNPACK_PERM = np.empty((D_IN,), np.int64)
for _g in range(D_IN // 32):
  for _k in range(16):
    UNPACK_PERM[32 * _g + _k] = 32 * _g + 2 * _k
    UNPACK_PERM[32 * _g + 16 + _k] = 32 * _g + 2 * _k + 1


def _sc_aggregate(tbl, srcr, dstr, wts):
  mesh = plsc.VectorSubcoreMesh(core_axis_name="c", subcore_axis_name="s")

  @functools.partial(
      pl.kernel,
      mesh=mesh,
      compiler_params=pltpu.CompilerParams(use_tc_tiling_on_sc=False,
                                           needs_layout_passes=False),
      out_type=jax.ShapeDtypeStruct((NUM_CORES, N_NODES, D_ACC), jnp.float32),
      scratch_types=[
          pltpu.VMEM((3, 2, BLK, CHUNK), jnp.int32),
          pltpu.VMEM((3, BLK, CHUNK), jnp.float32),
          pltpu.VMEM((BLK, CHUNK, D_IN), jnp.bfloat16),
          pltpu.VMEM((2, CHUNK, D_ACC), jnp.float32),
          pltpu.VMEM_SHARED((N_NODES, D_ACC), jnp.float32),
          pltpu.SemaphoreType.DMA((BLK,)),
          pltpu.SemaphoreType.DMA((2,)),
          pltpu.SemaphoreType.DMA((3,)),
          pltpu.SemaphoreType.DMA((3,)),
      ],
  )
  def k(tbl_h, src_h, dst_h, wts_h, out_h, idx_v, w_v, gbuf, sbuf,
        acc, gsem, ssem, stsem, wstsem):
    c = lax.axis_index("c")
    s = lax.axis_index("s")
    base = s * ROWS_PER_TILE

    def stage(blk_i, t3):
      pltpu.async_copy(src_h.at[s, pl.ds(blk_i * BLK, BLK)],
                       idx_v.at[t3, 0], stsem.at[t3])
      pltpu.async_copy(dst_h.at[s, pl.ds(blk_i * BLK, BLK)],
                       idx_v.at[t3, 1], stsem.at[t3])
      pltpu.async_copy(wts_h.at[s, pl.ds(blk_i * BLK, BLK)],
                       w_v.at[t3], wstsem.at[t3])

    def stage_wait(blk_i, t3):
      pltpu.make_async_copy(src_h.at[s, pl.ds(blk_i * BLK, BLK)],
                            idx_v.at[t3, 0], stsem.at[t3]).wait()
      pltpu.make_async_copy(dst_h.at[s, pl.ds(blk_i * BLK, BLK)],
                            idx_v.at[t3, 1], stsem.at[t3]).wait()
      pltpu.make_async_copy(wts_h.at[s, pl.ds(blk_i * BLK, BLK)],
                            w_v.at[t3], wstsem.at[t3]).wait()

    def zrow(i, carry):
      for d in range(D_ACC // 16):
        sbuf[0, i, pl.ds(d * 16, 16)] = jnp.zeros((16,), jnp.float32)
      return carry
    lax.fori_loop(0, CHUNK, zrow, 0)
    for t in range(ROWS_PER_TILE // CHUNK):
      pltpu.sync_copy(sbuf.at[0], acc.at[pl.ds(base + t * CHUNK, CHUNK)])
    rem = ROWS_PER_TILE % CHUNK
    if rem:
      pltpu.sync_copy(sbuf.at[0, pl.ds(0, rem)],
                      acc.at[pl.ds(base + ROWS_PER_TILE - rem, rem)])
    plsc.subcore_barrier()

    stage(0, 0)
    stage(1, 1)
    stage(2, 2)
    stage_wait(0, 0)
    for q in range(BLK):
      pltpu.async_copy(tbl_h.at[idx_v.at[0, c, q]], gbuf.at[q], gsem.at[q])

    def body(i, carry):
      b3 = lax.rem(i, 3)
      n3 = lax.rem(i + 1, 3)
      p3 = lax.rem(i + 2, 3)

      for q in range(BLK):
        sp = q % 2
        pltpu.make_async_copy(tbl_h.at[idx_v.at[b3, c, q]], gbuf.at[q],
                              gsem.at[q]).wait()

        if q >= 2:
          pltpu.make_async_copy(sbuf.at[sp], acc.at[idx_v.at[b3, c, q]],
                                ssem.at[sp]).wait()
        else:
          @pl.when(i > 0)
          def _():
            pltpu.make_async_copy(sbuf.at[sp], acc.at[idx_v.at[b3, c, q]],
                                  ssem.at[sp]).wait()

        def mul(g, carry2):
          wv = w_v[b3, q, pl.ds(g * 16, 16)]
          for e16 in range(16):
            wsc = wv[e16]
            row = g * 16 + e16
            wsplat = jnp.broadcast_to(wsc, (16,))
            wb32 = plsc.pack(wsplat, wsplat,
                             format=plsc.PackFormat.INTERLEAVED)
            for h in range(D_IN // 32):
              v32 = gbuf[q, row, pl.ds(h * 32, 32)] * wb32
              va, vb = plsc.unpack(v32, format=plsc.PackFormat.INTERLEAVED)
              sbuf[sp, row, pl.ds(h * 32, 16)] = va
              sbuf[sp, row, pl.ds(h * 32 + 16, 16)] = vb
            sbuf[sp, row, pl.ds(D_IN, 16)] = wsplat
          return carry2
        lax.fori_loop(0, CHUNK // 16, mul, 0)

        pltpu.async_copy(sbuf.at[sp], acc.at[idx_v.at[b3, 1 - c, q]],
                         ssem.at[sp], add=True)

        if q == 0:
          @pl.when(i + 1 < NITER)
          def _():
            stage_wait(i + 1, n3)

        if q == 1:
          @pl.when(i + 2 < NITER)
          def _():
            stage(i + 2, p3)

        @pl.when(i + 1 < NITER)
        def _():
          pltpu.async_copy(tbl_h.at[idx_v.at[n3, c, q]], gbuf.at[q],
                           gsem.at[q])
      return carry
    lax.fori_loop(0, NITER, body, 0)

    for sp in (0, 1):
      pltpu.make_async_copy(sbuf.at[sp], acc.at[idx_v.at[0, 0, 0]],
                            ssem.at[sp]).wait()
    plsc.subcore_barrier()
    pltpu.sync_copy(acc.at[pl.ds(base, ROWS_PER_TILE)],
                    out_h.at[c, pl.ds(base, ROWS_PER_TILE)])

  return k(tbl, srcr, dstr, wts)


def _tc_dense(agg, feat, w1t, w2t, a1, a2, whht, bih, bhh):
  blk = 1000
  grid = (N_NODES // blk,)

  def body(agg_ref, feat_ref, w1_ref, w2_ref, a1_ref, a2_ref, whh_ref,
           bih_ref, bhh_ref, out_ref):
    m1 = agg_ref[0]
    m2 = agg_ref[1]
    ws1 = m1[:, D_IN:D_IN + 1]
    ws2 = m2[:, D_IN:D_IN + 1]
    neigh1 = jnp.where(ws1 > 0, m1[:, :D_IN] / jnp.where(ws1 > 0, ws1, 1.0),
                       0.0)
    neigh2 = jnp.where(ws2 > 0, m2[:, :D_IN] / jnp.where(ws2 > 0, ws2, 1.0),
                       0.0)
    dot = functools.partial(jnp.dot, preferred_element_type=jnp.float32)
    n1 = dot(neigh1, w1_ref[...])
    n2 = dot(neigh2, w2_ref[...])
    gi = dot(n1, a1_ref[...]) + dot(n2, a2_ref[...]) + bih_ref[...]
    ft = feat_ref[...]
    gh = dot(ft, whh_ref[...]) + bhh_ref[...]
    r = jax.nn.sigmoid(gi[:, :D_IN] + gh[:, :D_IN])
    z = jax.nn.sigmoid(gi[:, D_IN:2 * D_IN] + gh[:, D_IN:2 * D_IN])
    n = jnp.tanh(gi[:, 2 * D_IN:] + r * gh[:, 2 * D_IN:])
    out_ref[...] = (1.0 - z) * n + z * ft

  return pl.pallas_call(
      body,
      grid=grid,
      in_specs=[
          pl.BlockSpec((NUM_CORES, blk, D_ACC), lambda i: (0, i, 0)),
          pl.BlockSpec((blk, D_IN), lambda i: (i, 0)),
          pl.BlockSpec((D_IN, D_IN), lambda i: (0, 0)),
          pl.BlockSpec((D_IN, D_IN), lambda i: (0, 0)),
          pl.BlockSpec((D_IN, 3 * D_IN), lambda i: (0, 0)),
          pl.BlockSpec((D_IN, 3 * D_IN), lambda i: (0, 0)),
          pl.BlockSpec((D_IN, 3 * D_IN), lambda i: (0, 0)),
          pl.BlockSpec((1, 3 * D_IN), lambda i: (0, 0)),
          pl.BlockSpec((1, 3 * D_IN), lambda i: (0, 0)),
      ],
      out_specs=pl.BlockSpec((blk, D_IN), lambda i: (i, 0)),
      out_shape=jax.ShapeDtypeStruct((N_NODES, D_IN), jnp.float32),
  )(agg, feat, w1t, w2t, a1, a2, whht, bih, bhh)


@jax.jit
def kernel(feat, edge_index, edge_weight, W1, W2, W_ih, W_hh, b_ih, b_hh):
  pad = E_PAD - N_EDGES
  src = jnp.concatenate([edge_index[0], jnp.zeros((pad,), jnp.int32)])
  dst = jnp.concatenate([edge_index[1], jnp.zeros((pad,), jnp.int32)])
  w = jnp.concatenate([edge_weight, jnp.zeros((pad,), jnp.float32)])
  srcr = src.reshape(NUM_TILES, K_CHUNKS, CHUNK)
  dstr = dst.reshape(NUM_TILES, K_CHUNKS, CHUNK)
  wts = w.reshape(NUM_TILES, K_CHUNKS, CHUNK)
  tbl = feat.astype(jnp.bfloat16)

  tbl, srcr, dstr, wts = lax.optimization_barrier((tbl, srcr, dstr, wts))
  agg = _sc_aggregate(tbl, srcr, dstr, wts)

  w1t = W1.T[UNPACK_PERM]
  w2t = W2.T[UNPACK_PERM]
  wiht = W_ih.T
  a1 = wiht[:D_IN]
  a2 = wiht[D_IN:]
  whht = W_hh.T
  bih = b_ih.reshape(1, 3 * D_IN)
  bhh = b_hh.reshape(1, 3 * D_IN)
  return _tc_dense(agg, feat, w1t, w2t, a1, a2, whht, bih, bhh)


# speedup vs baseline: 1.1000x; 1.0103x over previous
import functools

import jax
import jax.numpy as jnp
import numpy as np
from jax import lax
from jax.experimental import pallas as pl
from jax.experimental.pallas import tpu as pltpu
from jax.experimental.pallas import tpu_sc as plsc

N_NODES = 10000
D_IN = 128
D_ACC = 144
N_EDGES = 320000
NUM_CORES = 2
NUM_TILES = 16
CHUNK = 64
BLK = 4
K_CHUNKS = 320
NITER = K_CHUNKS // BLK
PT = K_CHUNKS * CHUNK
E_PAD = NUM_TILES * PT
ROWS_PER_TILE = N_NODES // NUM_TILES

UNPACK_PERM = np.empty((D_IN,), np.int64)
for _g in range(D_IN // 32):
  for _k in range(16):
    UNPACK_PERM[32 * _g + _k] = 32 * _g + 2 * _k
    UNPACK_PERM[32 * _g + 16 + _k] = 32 * _g + 2 * _k + 1


def _sc_aggregate(tbl, srcr, dstr, wts):
  mesh = plsc.VectorSubcoreMesh(core_axis_name="c", subcore_axis_name="s")

  @functools.partial(
      pl.kernel,
      mesh=mesh,
      compiler_params=pltpu.CompilerParams(use_tc_tiling_on_sc=False,
                                           needs_layout_passes=False),
      out_type=jax.ShapeDtypeStruct((NUM_CORES, N_NODES, D_ACC), jnp.float32),
      scratch_types=[
          pltpu.VMEM((3, 2, BLK, CHUNK), jnp.int32),
          pltpu.VMEM((3, BLK, CHUNK), jnp.float32),
          pltpu.VMEM((BLK, CHUNK, D_IN), jnp.bfloat16),
          pltpu.VMEM((2, CHUNK, D_ACC), jnp.float32),
          pltpu.VMEM_SHARED((N_NODES, D_ACC), jnp.float32),
          pltpu.SemaphoreType.DMA((BLK,)),
          pltpu.SemaphoreType.DMA((2,)),
          pltpu.SemaphoreType.DMA((3,)),
          pltpu.SemaphoreType.DMA((3,)),
      ],
  )
  def k(tbl_h, src_h, dst_h, wts_h, out_h, idx_v, w_v, gbuf, sbuf,
        acc, gsem, ssem, stsem, wstsem):
    c = lax.axis_index("c")
    s = lax.axis_index("s")
    base = s * ROWS_PER_TILE

    def stage(blk_i, t3):
      pltpu.async_copy(src_h.at[s, pl.ds(blk_i * BLK, BLK)],
                       idx_v.at[t3, 0], stsem.at[t3])
      pltpu.async_copy(dst_h.at[s, pl.ds(blk_i * BLK, BLK)],
                       idx_v.at[t3, 1], stsem.at[t3])
      pltpu.async_copy(wts_h.at[s, pl.ds(blk_i * BLK, BLK)],
                       w_v.at[t3], wstsem.at[t3])

    def stage_wait(blk_i, t3):
      pltpu.make_async_copy(src_h.at[s, pl.ds(blk_i * BLK, BLK)],
                            idx_v.at[t3, 0], stsem.at[t3]).wait()
      pltpu.make_async_copy(dst_h.at[s, pl.ds(blk_i * BLK, BLK)],
                            idx_v.at[t3, 1], stsem.at[t3]).wait()
      pltpu.make_async_copy(wts_h.at[s, pl.ds(blk_i * BLK, BLK)],
                            w_v.at[t3], wstsem.at[t3]).wait()

    def zrow(i, carry):
      for d in range(D_ACC // 16):
        sbuf[0, i, pl.ds(d * 16, 16)] = jnp.zeros((16,), jnp.float32)
      return carry
    lax.fori_loop(0, CHUNK, zrow, 0)
    for t in range(ROWS_PER_TILE // CHUNK):
      pltpu.sync_copy(sbuf.at[0], acc.at[pl.ds(base + t * CHUNK, CHUNK)])
    rem = ROWS_PER_TILE % CHUNK
    if rem:
      pltpu.sync_copy(sbuf.at[0, pl.ds(0, rem)],
                      acc.at[pl.ds(base + ROWS_PER_TILE - rem, rem)])
    plsc.subcore_barrier()

    stage(0, 0)
    stage(1, 1)
    stage(2, 2)
    stage_wait(0, 0)
    for q in range(BLK):
      pltpu.async_copy(tbl_h.at[idx_v.at[0, c, q]], gbuf.at[q], gsem.at[q])

    def body(i, carry):
      b3 = lax.rem(i, 3)
      n3 = lax.rem(i + 1, 3)
      p3 = lax.rem(i + 2, 3)

      for q in range(BLK):
        sp = q % 2
        pltpu.make_async_copy(tbl_h.at[idx_v.at[b3, c, q]], gbuf.at[q],
                              gsem.at[q]).wait()

        if q >= 2:
          pltpu.make_async_copy(sbuf.at[sp], acc.at[idx_v.at[b3, c, q]],
                                ssem.at[sp]).wait()
        else:
          @pl.when(i > 0)
          def _():
            pltpu.make_async_copy(sbuf.at[sp], acc.at[idx_v.at[b3, c, q]],
                                  ssem.at[sp]).wait()

        def mul(g, carry2):
          wv = w_v[b3, q, pl.ds(g * 16, 16)]
          for e16 in range(16):
            wsc = wv[e16]
            row = g * 16 + e16
            wsplat = jnp.broadcast_to(wsc, (16,))
            wb32 = plsc.pack(wsplat, wsplat,
                             format=plsc.PackFormat.INTERLEAVED)
            for h in range(D_IN // 32):
              v32 = gbuf[q, row, pl.ds(h * 32, 32)] * wb32
              va, vb = plsc.unpack(v32, format=plsc.PackFormat.INTERLEAVED)
              sbuf[sp, row, pl.ds(h * 32, 16)] = va
              sbuf[sp, row, pl.ds(h * 32 + 16, 16)] = vb
            sbuf[sp, row, pl.ds(D_IN, 16)] = wsplat
          return carry2
        lax.fori_loop(0, CHUNK // 16, mul, 0)

        pltpu.async_copy(sbuf.at[sp], acc.at[idx_v.at[b3, 1 - c, q]],
                         ssem.at[sp], add=True)

        if q == 0:
          @pl.when(i + 1 < NITER)
          def _():
            stage_wait(i + 1, n3)

        if q == 1:
          @pl.when(i + 2 < NITER)
          def _():
            stage(i + 2, p3)

        @pl.when(i + 1 < NITER)
        def _():
          pltpu.async_copy(tbl_h.at[idx_v.at[n3, c, q]], gbuf.at[q],
                           gsem.at[q])
      return carry
    lax.fori_loop(0, NITER, body, 0)

    for sp in (0, 1):
      pltpu.make_async_copy(sbuf.at[sp], acc.at[idx_v.at[0, 0, 0]],
                            ssem.at[sp]).wait()
    plsc.subcore_barrier()
    pltpu.sync_copy(acc.at[pl.ds(base, ROWS_PER_TILE)],
                    out_h.at[c, pl.ds(base, ROWS_PER_TILE)])

  return k(tbl, srcr, dstr, wts)


def _tc_gh(feat, whht, bhh):
  blk = 1000
  grid = (N_NODES // blk,)

  def body(feat_ref, whh_ref, bhh_ref, out_ref):
    dot = functools.partial(jnp.dot, preferred_element_type=jnp.float32)
    out_ref[...] = dot(feat_ref[...], whh_ref[...]) + bhh_ref[...]

  return pl.pallas_call(
      body,
      grid=grid,
      in_specs=[
          pl.BlockSpec((blk, D_IN), lambda i: (i, 0)),
          pl.BlockSpec((D_IN, 3 * D_IN), lambda i: (0, 0)),
          pl.BlockSpec((1, 3 * D_IN), lambda i: (0, 0)),
      ],
      out_specs=pl.BlockSpec((blk, 3 * D_IN), lambda i: (i, 0)),
      out_shape=jax.ShapeDtypeStruct((N_NODES, 3 * D_IN), jnp.float32),
  )(feat, whht, bhh)


def _tc_dense(agg, feat, gh_in, w1t, w2t, a1, a2, bih):
  blk = 1000
  grid = (N_NODES // blk,)

  def body(agg_ref, feat_ref, gh_ref, w1_ref, w2_ref, a1_ref, a2_ref,
           bih_ref, out_ref):
    m1 = agg_ref[0]
    m2 = agg_ref[1]
    ws1 = m1[:, D_IN:D_IN + 1]
    ws2 = m2[:, D_IN:D_IN + 1]
    neigh1 = jnp.where(ws1 > 0, m1[:, :D_IN] / jnp.where(ws1 > 0, ws1, 1.0),
                       0.0)
    neigh2 = jnp.where(ws2 > 0, m2[:, :D_IN] / jnp.where(ws2 > 0, ws2, 1.0),
                       0.0)
    dot = functools.partial(jnp.dot, preferred_element_type=jnp.float32)
    n1 = dot(neigh1, w1_ref[...])
    n2 = dot(neigh2, w2_ref[...])
    gi = dot(n1, a1_ref[...]) + dot(n2, a2_ref[...]) + bih_ref[...]
    ft = feat_ref[...]
    gh = gh_ref[...]
    r = jax.nn.sigmoid(gi[:, :D_IN] + gh[:, :D_IN])
    z = jax.nn.sigmoid(gi[:, D_IN:2 * D_IN] + gh[:, D_IN:2 * D_IN])
    n = jnp.tanh(gi[:, 2 * D_IN:] + r * gh[:, 2 * D_IN:])
    out_ref[...] = (1.0 - z) * n + z * ft

  return pl.pallas_call(
      body,
      grid=grid,
      in_specs=[
          pl.BlockSpec((NUM_CORES, blk, D_ACC), lambda i: (0, i, 0)),
          pl.BlockSpec((blk, D_IN), lambda i: (i, 0)),
          pl.BlockSpec((blk, 3 * D_IN), lambda i: (i, 0)),
          pl.BlockSpec((D_IN, D_IN), lambda i: (0, 0)),
          pl.BlockSpec((D_IN, D_IN), lambda i: (0, 0)),
          pl.BlockSpec((D_IN, 3 * D_IN), lambda i: (0, 0)),
          pl.BlockSpec((D_IN, 3 * D_IN), lambda i: (0, 0)),
          pl.BlockSpec((1, 3 * D_IN), lambda i: (0, 0)),
      ],
      out_specs=pl.BlockSpec((blk, D_IN), lambda i: (i, 0)),
      out_shape=jax.ShapeDtypeStruct((N_NODES, D_IN), jnp.float32),
  )(agg, feat, gh_in, w1t, w2t, a1, a2, bih)


@jax.jit
def kernel(feat, edge_index, edge_weight, W1, W2, W_ih, W_hh, b_ih, b_hh):
  pad = E_PAD - N_EDGES
  src = jnp.concatenate([edge_index[0], jnp.zeros((pad,), jnp.int32)])
  dst = jnp.concatenate([edge_index[1], jnp.zeros((pad,), jnp.int32)])
  w = jnp.concatenate([edge_weight, jnp.zeros((pad,), jnp.float32)])
  srcr = src.reshape(NUM_TILES, K_CHUNKS, CHUNK)
  dstr = dst.reshape(NUM_TILES, K_CHUNKS, CHUNK)
  wts = w.reshape(NUM_TILES, K_CHUNKS, CHUNK)
  tbl = feat.astype(jnp.bfloat16)

  tbl, srcr, dstr, wts = lax.optimization_barrier((tbl, srcr, dstr, wts))
  whht = W_hh.T
  bhh = b_hh.reshape(1, 3 * D_IN)
  gh = _tc_gh(feat, whht, bhh)
  agg = _sc_aggregate(tbl, srcr, dstr, wts)

  w1t = W1.T[UNPACK_PERM]
  w2t = W2.T[UNPACK_PERM]
  wiht = W_ih.T
  a1 = wiht[:D_IN]
  a2 = wiht[D_IN:]
  bih = b_ih.reshape(1, 3 * D_IN)
  return _tc_dense(agg, feat, gh, w1t, w2t, a1, a2, bih)
